# Initial kernel scaffold; baseline (speedup 1.0000x reference)
import jax
import jax.numpy as jnp
from jax.experimental import pallas as pl


def kernel(x_seq, edge_index_seq, batch_seq, W0, b0, gamma0, beta0, alpha0, W1, b1, gamma1, beta1, alpha1, W2, b2, gamma2, beta2, alpha2):
    raise NotImplementedError("write your pallas kernel here")


# speedup vs baseline: 15.8219x; 15.8219x over previous
import functools
import jax
import jax.numpy as jnp
from jax import lax
from jax.experimental import pallas as pl
from jax.experimental.pallas import tpu as pltpu
from jax.experimental.pallas import tpu_sc as plsc

T, N, E, F, G = 4, 10000, 320000, 128, 16
NTILES = 16
NCORES = 2
WIN = 128
UNROLL = 8
IDXC = 16
EPAD = 32 * WIN * 80
KW = EPAD // (NCORES * NTILES * WIN)
ND = 80
NP = 10240
ROWS_T = NP // NTILES
NB = 1024
NBLK = NP // NB
KD = T * EPAD // (NCORES * NTILES * WIN)
ACC = N + ND
DEGL = T * ACC


def _sc_mesh():
    return plsc.VectorSubcoreMesh(core_axis_name="c", subcore_axis_name="s")


def _deg_body(idx_hbm, zeros_hbm, out_hbm, idx_v, ones_v, bounce, acc_sp):
    c = lax.axis_index("c")
    s = lax.axis_index("s")
    for i in range(8):
        ones_v[pl.ds(i * 16, 16)] = jnp.ones((16,), jnp.float32)
    seg = DEGL // NTILES
    pltpu.sync_copy(zeros_hbm.at[pl.ds(s * seg, seg)], bounce)
    pltpu.sync_copy(bounce, acc_sp.at[pl.ds(s * seg, seg)])
    pltpu.sync_copy(idx_hbm.at[c, s], idx_v)
    plsc.subcore_barrier()

    def chunk(k, carry):
        for j in range(UNROLL):
            pltpu.sync_copy(ones_v, acc_sp.at[idx_v.at[k * UNROLL + j]], add=True)
        return carry

    lax.fori_loop(0, KD // UNROLL, chunk, 0)
    plsc.subcore_barrier()
    pltpu.sync_copy(acc_sp.at[pl.ds(s * seg, seg)], bounce)
    pltpu.sync_copy(bounce, out_hbm.at[pl.ds(c * DEGL + s * seg, seg)])


def _deg_counts(idx_w, zeros_flat):
    seg = DEGL // NTILES
    k = pl.kernel(
        _deg_body,
        out_type=jax.ShapeDtypeStruct((NCORES * DEGL,), jnp.float32),
        mesh=_sc_mesh(),
        scratch_types=[
            pltpu.VMEM((KD, WIN), jnp.int32),
            pltpu.VMEM((WIN,), jnp.float32),
            pltpu.VMEM((seg,), jnp.float32),
            pltpu.VMEM_SHARED((DEGL,), jnp.float32),
        ],
    )
    return k(idx_w, zeros_flat)


def _scat_body(z_hbm, src_hbm, dst_hbm, out_hbm, src_v, dst_v, buf, gsem,
               acc_sp):
    c = lax.axis_index("c")
    s = lax.axis_index("s")
    nstage = ROWS_T // WIN
    for t in range(T):
        def stage_in(r, carry):
            rows = pl.ds(t * NP + s * ROWS_T + r * WIN, WIN)
            pltpu.sync_copy(z_hbm.at[rows], buf)
            pltpu.sync_copy(buf, acc_sp.at[pl.ds(s * ROWS_T + r * WIN, WIN)])
            return carry

        lax.fori_loop(0, nstage, stage_in, 0)
        plsc.subcore_barrier()

        def idx_chunk(kc, carry):
            pltpu.sync_copy(src_hbm.at[t, c, s, pl.ds(kc * IDXC, IDXC)], src_v)
            pltpu.sync_copy(dst_hbm.at[t, c, s, pl.ds(kc * IDXC, IDXC)], dst_v)

            def chunk(k, carry2):
                for j in range(UNROLL):
                    w = k * UNROLL + j
                    pltpu.async_copy(z_hbm.at[src_v.at[w]], buf, gsem).wait()
                    pltpu.sync_copy(buf, acc_sp.at[dst_v.at[w]], add=True)
                return carry2

            lax.fori_loop(0, IDXC // UNROLL, chunk, 0)
            return carry

        lax.fori_loop(0, KW // IDXC, idx_chunk, 0)
        plsc.subcore_barrier()

        def stage_out(r, carry):
            rows = pl.ds(s * ROWS_T + r * WIN, WIN)
            pltpu.sync_copy(acc_sp.at[rows], buf)
            pltpu.sync_copy(buf, out_hbm.at[c, t, rows])
            return carry

        lax.fori_loop(0, nstage, stage_out, 0)
        plsc.subcore_barrier()


def _propagate(z_all, src_g, dst_w):
    k = pl.kernel(
        _scat_body,
        out_type=jax.ShapeDtypeStruct((NCORES, T, NP, F), jnp.float32),
        mesh=_sc_mesh(),
        scratch_types=[
            pltpu.VMEM((IDXC, WIN), jnp.int32),
            pltpu.VMEM((IDXC, WIN), jnp.int32),
            pltpu.VMEM((WIN, F), jnp.float32),
            pltpu.SemaphoreType.DMA,
            pltpu.VMEM_SHARED((NP, F), jnp.float32),
        ],
    )
    return k(z_all.reshape(T * NP, F), src_g, dst_w)


def _prep_body(cnt_ref, x_ref, dinv_ref, z_ref):
    n = pl.program_id(1)
    rows = n * NB + lax.broadcasted_iota(jnp.int32, (NB, 1), 0)
    c = cnt_ref[0, 0] + cnt_ref[1, 0]
    dinv = jnp.where(rows < N, lax.rsqrt(1.0 + c), 0.0)
    dinv_ref[0] = dinv
    z_ref[0] = dinv * x_ref[0]


def _prep(cnt2, x_pad):
    return pl.pallas_call(
        _prep_body,
        grid=(T, NBLK),
        in_specs=[
            pl.BlockSpec((2, 1, NB, 1), lambda t, n: (0, t, n, 0)),
            pl.BlockSpec((1, NB, F), lambda t, n: (t, n, 0)),
        ],
        out_specs=[
            pl.BlockSpec((1, NB, 1), lambda t, n: (t, n, 0)),
            pl.BlockSpec((1, NB, F), lambda t, n: (t, n, 0)),
        ],
        out_shape=[
            jax.ShapeDtypeStruct((T, NP, 1), jnp.float32),
            jax.ShapeDtypeStruct((T, NP, F), jnp.float32),
        ],
    )(cnt2, x_pad)


def _passA_body(s_ref, z_ref, dinv_ref, bht_ref, w_ref, b_ref,
                h_ref, s1_ref, s2_ref, cnt_ref):
    n = pl.program_id(1)
    dinv = dinv_ref[0]
    stot = s_ref[0, 0] + s_ref[1, 0] - z_ref[0]
    xin = jnp.where(dinv > 0, stot, 0.0) * dinv
    h = jnp.dot(xin, w_ref[...], preferred_element_type=jnp.float32) + b_ref[...]
    h_ref[0] = h
    bht = bht_ref[0]
    p1 = jnp.dot(bht, h, preferred_element_type=jnp.float32)
    p2 = jnp.dot(bht, h * h, preferred_element_type=jnp.float32)
    pc = jnp.sum(bht, axis=1, keepdims=True) * jnp.ones((1, 8), jnp.float32)

    @pl.when(n == 0)
    def _():
        s1_ref[0] = p1
        s2_ref[0] = p2
        cnt_ref[0] = pc

    @pl.when(n != 0)
    def _():
        s1_ref[0] += p1
        s2_ref[0] += p2
        cnt_ref[0] += pc


def _passA(s_all, z, dinv, bht, w, b2d, ho):
    return pl.pallas_call(
        _passA_body,
        grid=(T, NBLK),
        in_specs=[
            pl.BlockSpec((2, 1, NB, F), lambda t, n: (0, t, n, 0)),
            pl.BlockSpec((1, NB, F), lambda t, n: (t, n, 0)),
            pl.BlockSpec((1, NB, 1), lambda t, n: (t, n, 0)),
            pl.BlockSpec((1, G, NB), lambda t, n: (t, 0, n)),
            pl.BlockSpec((F, ho), lambda t, n: (0, 0)),
            pl.BlockSpec((1, ho), lambda t, n: (0, 0)),
        ],
        out_specs=[
            pl.BlockSpec((1, NB, ho), lambda t, n: (t, n, 0)),
            pl.BlockSpec((1, G, ho), lambda t, n: (t, 0, 0)),
            pl.BlockSpec((1, G, ho), lambda t, n: (t, 0, 0)),
            pl.BlockSpec((1, G, 8), lambda t, n: (t, 0, 0)),
        ],
        out_shape=[
            jax.ShapeDtypeStruct((T, NP, ho), jnp.float32),
            jax.ShapeDtypeStruct((T, G, ho), jnp.float32),
            jax.ShapeDtypeStruct((T, G, ho), jnp.float32),
            jax.ShapeDtypeStruct((T, G, 8), jnp.float32),
        ],
    )(s_all, z, dinv, bht, w, b2d)


def _passB_mid_body(h_ref, mean_ref, rstd_ref, dinv_ref, bht_ref,
                    gm_ref, bt_ref, al_ref, z_ref):
    h = h_ref[0]
    bht = bht_ref[0]
    mb = lax.dot_general(bht, mean_ref[0], (((0,), (0,)), ((), ())),
                         preferred_element_type=jnp.float32)
    rb = lax.dot_general(bht, rstd_ref[0], (((0,), (0,)), ((), ())),
                         preferred_element_type=jnp.float32)
    x = jnp.maximum(gm_ref[...] * (h - al_ref[...] * mb) * rb + bt_ref[...], 0.0)
    z_ref[0] = dinv_ref[0] * x


def _passB_mid(h, mean, rstd, dinv, bht, gm2, bt2, al2, ho):
    return pl.pallas_call(
        _passB_mid_body,
        grid=(T, NBLK),
        in_specs=[
            pl.BlockSpec((1, NB, ho), lambda t, n: (t, n, 0)),
            pl.BlockSpec((1, G, ho), lambda t, n: (t, 0, 0)),
            pl.BlockSpec((1, G, ho), lambda t, n: (t, 0, 0)),
            pl.BlockSpec((1, NB, 1), lambda t, n: (t, n, 0)),
            pl.BlockSpec((1, G, NB), lambda t, n: (t, 0, n)),
            pl.BlockSpec((1, ho), lambda t, n: (0, 0)),
            pl.BlockSpec((1, ho), lambda t, n: (0, 0)),
            pl.BlockSpec((1, ho), lambda t, n: (0, 0)),
        ],
        out_specs=[pl.BlockSpec((1, NB, F), lambda t, n: (t, n, 0))],
        out_shape=[jax.ShapeDtypeStruct((T, NP, F), jnp.float32)],
    )(h, mean, rstd, dinv, bht, gm2, bt2, al2)[0]


def _passB_last_body(h_ref, mean_ref, rstd_ref, bht_ref,
                     gm_ref, bt_ref, al_ref, psum_ref, pmax_ref):
    n = pl.program_id(1)
    h = h_ref[0]
    bht = bht_ref[0]
    mb = lax.dot_general(bht, mean_ref[0], (((0,), (0,)), ((), ())),
                         preferred_element_type=jnp.float32)
    rb = lax.dot_general(bht, rstd_ref[0], (((0,), (0,)), ((), ())),
                         preferred_element_type=jnp.float32)
    x = jnp.maximum(gm_ref[...] * (h - al_ref[...] * mb) * rb + bt_ref[...], 0.0)
    ps = jnp.dot(bht, x, preferred_element_type=jnp.float32)
    neg = jnp.float32(-jnp.inf)
    pm = jnp.stack([jnp.max(jnp.where(bht[g][:, None] > 0, x, neg), axis=0)
                    for g in range(G)], axis=0)

    @pl.when(n == 0)
    def _():
        psum_ref[0] = ps
        pmax_ref[0] = pm

    @pl.when(n != 0)
    def _():
        psum_ref[0] += ps
        pmax_ref[0] = jnp.maximum(pmax_ref[0], pm)


def _passB_last(h, mean, rstd, bht, gm2, bt2, al2, ho):
    return pl.pallas_call(
        _passB_last_body,
        grid=(T, NBLK),
        in_specs=[
            pl.BlockSpec((1, NB, ho), lambda t, n: (t, n, 0)),
            pl.BlockSpec((1, G, ho), lambda t, n: (t, 0, 0)),
            pl.BlockSpec((1, G, ho), lambda t, n: (t, 0, 0)),
            pl.BlockSpec((1, G, NB), lambda t, n: (t, 0, n)),
            pl.BlockSpec((1, ho), lambda t, n: (0, 0)),
            pl.BlockSpec((1, ho), lambda t, n: (0, 0)),
            pl.BlockSpec((1, ho), lambda t, n: (0, 0)),
        ],
        out_specs=[
            pl.BlockSpec((1, G, ho), lambda t, n: (t, 0, 0)),
            pl.BlockSpec((1, G, ho), lambda t, n: (t, 0, 0)),
        ],
        out_shape=[
            jax.ShapeDtypeStruct((T, G, ho), jnp.float32),
            jax.ShapeDtypeStruct((T, G, ho), jnp.float32),
        ],
    )(h, mean, rstd, bht, gm2, bt2, al2)


def _moments_to_norm(s1, s2, cnt8, alpha):
    cnt = cnt8[:, :, :1]
    mean = s1 / cnt
    eh2 = s2 / cnt
    var = eh2 - (alpha * (2.0 - alpha)) * mean * mean
    rstd = lax.rsqrt(var + 1e-5)
    return mean, rstd, cnt


def kernel(x_seq, edge_index_seq, batch_seq, W0, b0, gamma0, beta0, alpha0,
           W1, b1, gamma1, beta1, alpha1, W2, b2, gamma2, beta2, alpha2):
    f32 = jnp.float32
    pad_n = EPAD - E
    pad_rows = (jnp.arange(pad_n, dtype=jnp.int32) % ND)
    src = jnp.concatenate(
        [edge_index_seq[:, 0].astype(jnp.int32),
         jnp.broadcast_to(pad_rows, (T, pad_n))], axis=1)
    dst = jnp.concatenate(
        [edge_index_seq[:, 1].astype(jnp.int32),
         jnp.broadcast_to(N + pad_rows, (T, pad_n))], axis=1)
    dst_w = dst.reshape(T, NCORES, NTILES, KW, WIN)
    t_off = (jnp.arange(T, dtype=jnp.int32) * NP)[:, None, None, None, None]
    src_g = src.reshape(T, NCORES, NTILES, KW, WIN) + t_off
    toff = (jnp.arange(T, dtype=jnp.int32) * ACC)[:, None]
    deg_idx = (dst + toff).reshape(NCORES, NTILES, KD, WIN)
    zeros_flat = jnp.zeros((DEGL,), f32)

    bht = (batch_seq[:, None, :] == jnp.arange(G, dtype=batch_seq.dtype)[None, :, None]
           ).astype(f32)
    bht = jnp.pad(bht, ((0, 0), (0, 0), (0, NP - N)))
    x_pad = jnp.pad(x_seq, ((0, 0), (0, NP - N), (0, 0)))

    parts = _deg_counts(deg_idx, zeros_flat)
    cnt2 = parts.reshape(NCORES, T, ACC)[:, :, :N]
    cnt2 = jnp.pad(cnt2, ((0, 0), (0, 0), (0, NP - N)))[..., None]

    dinv, z = _prep(cnt2, x_pad)

    params = [(W0, b0, gamma0, beta0, alpha0),
              (W1, b1, gamma1, beta1, alpha1),
              (W2, b2, gamma2, beta2, alpha2)]
    for li, (W, b, gm, bt, al) in enumerate(params):
        ho = W.shape[1]
        s_all = _propagate(z, src_g, dst_w)
        h, s1, s2, cnt8 = _passA(s_all, z, dinv, bht, W, b.reshape(1, ho), ho)
        mean, rstd, cnt = _moments_to_norm(s1, s2, cnt8, al)
        if li < 2:
            z = _passB_mid(h, mean, rstd, dinv, bht, gm.reshape(1, ho),
                           bt.reshape(1, ho), al.reshape(1, ho), ho)
        else:
            psum, pmax = _passB_last(h, mean, rstd, bht, gm.reshape(1, ho),
                                     bt.reshape(1, ho), al.reshape(1, ho), ho)
    out = jnp.concatenate([psum / cnt, pmax], axis=-1)
    return jnp.mean(out, axis=0)


# speedup vs baseline: 20.3770x; 1.2879x over previous
import functools
import jax
import jax.numpy as jnp
from jax import lax
from jax.experimental import pallas as pl
from jax.experimental.pallas import tpu as pltpu
from jax.experimental.pallas import tpu_sc as plsc

T, N, E, F, G = 4, 10000, 320000, 128, 16
NTILES = 16
NCORES = 2
WIN = 128
UNROLL = 8
IDXC = 8
EPAD = 32 * WIN * 80
KW = EPAD // (NCORES * NTILES * WIN)
ND = 80
NP = 10240
ROWS_T = NP // NTILES
NB = 1024
NBLK = NP // NB
KD = T * EPAD // (NCORES * NTILES * WIN)
ACC = N + ND
DEGL = T * ACC


def _sc_mesh():
    return plsc.VectorSubcoreMesh(core_axis_name="c", subcore_axis_name="s")


def _deg_body(idx_hbm, zeros_hbm, out_hbm, idx_v, ones_v, bounce, acc_sp):
    c = lax.axis_index("c")
    s = lax.axis_index("s")
    for i in range(8):
        ones_v[pl.ds(i * 16, 16)] = jnp.ones((16,), jnp.float32)
    seg = DEGL // NTILES
    pltpu.sync_copy(zeros_hbm.at[pl.ds(s * seg, seg)], bounce)
    pltpu.sync_copy(bounce, acc_sp.at[pl.ds(s * seg, seg)])
    pltpu.sync_copy(idx_hbm.at[c, s], idx_v)
    plsc.subcore_barrier()

    def chunk(k, carry):
        for j in range(UNROLL):
            pltpu.sync_copy(ones_v, acc_sp.at[idx_v.at[k * UNROLL + j]], add=True)
        return carry

    lax.fori_loop(0, KD // UNROLL, chunk, 0)
    plsc.subcore_barrier()
    pltpu.sync_copy(acc_sp.at[pl.ds(s * seg, seg)], bounce)
    pltpu.sync_copy(bounce, out_hbm.at[pl.ds(c * DEGL + s * seg, seg)])


def _deg_counts(idx_w, zeros_flat):
    seg = DEGL // NTILES
    k = pl.kernel(
        _deg_body,
        out_type=jax.ShapeDtypeStruct((NCORES * DEGL,), jnp.float32),
        mesh=_sc_mesh(),
        scratch_types=[
            pltpu.VMEM((KD, WIN), jnp.int32),
            pltpu.VMEM((WIN,), jnp.float32),
            pltpu.VMEM((seg,), jnp.float32),
            pltpu.VMEM_SHARED((DEGL,), jnp.float32),
        ],
    )
    return k(idx_w, zeros_flat)


def _scat_body(z_hbm, src_hbm, dst_hbm, out_hbm, src_v, dst_v, buf0, buf1,
               gsem0, gsem1, acc_sp):
    c = lax.axis_index("c")
    s = lax.axis_index("s")
    bufs = (buf0, buf1)
    sems = (gsem0, gsem1)
    nstage = ROWS_T // WIN
    for t in range(T):
        def stage_in(r, carry):
            rows = pl.ds(t * NP + s * ROWS_T + r * WIN, WIN)
            pltpu.sync_copy(z_hbm.at[rows], buf0)
            pltpu.sync_copy(buf0, acc_sp.at[pl.ds(s * ROWS_T + r * WIN, WIN)])
            return carry

        lax.fori_loop(0, nstage, stage_in, 0)
        plsc.subcore_barrier()

        def idx_chunk(kc, carry):
            pltpu.sync_copy(src_hbm.at[t, c, s, pl.ds(kc * IDXC, IDXC)], src_v)
            pltpu.sync_copy(dst_hbm.at[t, c, s, pl.ds(kc * IDXC, IDXC)], dst_v)
            d = pltpu.async_copy(z_hbm.at[src_v.at[0]], buf0, gsem0)
            for j in range(IDXC - 1):
                nxt = pltpu.async_copy(z_hbm.at[src_v.at[j + 1]],
                                       bufs[(j + 1) % 2], sems[(j + 1) % 2])
                d.wait()
                pltpu.sync_copy(bufs[j % 2], acc_sp.at[dst_v.at[j]], add=True)
                d = nxt
            d.wait()
            pltpu.sync_copy(bufs[(IDXC - 1) % 2],
                            acc_sp.at[dst_v.at[IDXC - 1]], add=True)
            return carry

        lax.fori_loop(0, KW // IDXC, idx_chunk, 0)
        plsc.subcore_barrier()

        def stage_out(r, carry):
            rows = pl.ds(s * ROWS_T + r * WIN, WIN)
            pltpu.sync_copy(acc_sp.at[rows], buf0)
            pltpu.sync_copy(buf0, out_hbm.at[c, t, rows])
            return carry

        lax.fori_loop(0, nstage, stage_out, 0)
        plsc.subcore_barrier()


def _propagate(z_all, src_g, dst_w):
    k = pl.kernel(
        _scat_body,
        out_type=jax.ShapeDtypeStruct((NCORES, T, NP, F), jnp.float32),
        mesh=_sc_mesh(),
        scratch_types=[
            pltpu.VMEM((IDXC, WIN), jnp.int32),
            pltpu.VMEM((IDXC, WIN), jnp.int32),
            pltpu.VMEM((WIN, F), jnp.float32),
            pltpu.VMEM((WIN, F), jnp.float32),
            pltpu.SemaphoreType.DMA,
            pltpu.SemaphoreType.DMA,
            pltpu.VMEM_SHARED((NP, F), jnp.float32),
        ],
    )
    return k(z_all.reshape(T * NP, F), src_g, dst_w)


def _prep_body(cnt_ref, x_ref, dinv_ref, z_ref):
    n = pl.program_id(1)
    rows = n * NB + lax.broadcasted_iota(jnp.int32, (NB, 1), 0)
    c = cnt_ref[0, 0] + cnt_ref[1, 0]
    dinv = jnp.where(rows < N, lax.rsqrt(1.0 + c), 0.0)
    dinv_ref[0] = dinv
    z_ref[0] = dinv * x_ref[0]


def _prep(cnt2, x_pad):
    return pl.pallas_call(
        _prep_body,
        grid=(T, NBLK),
        in_specs=[
            pl.BlockSpec((2, 1, NB, 1), lambda t, n: (0, t, n, 0)),
            pl.BlockSpec((1, NB, F), lambda t, n: (t, n, 0)),
        ],
        out_specs=[
            pl.BlockSpec((1, NB, 1), lambda t, n: (t, n, 0)),
            pl.BlockSpec((1, NB, F), lambda t, n: (t, n, 0)),
        ],
        out_shape=[
            jax.ShapeDtypeStruct((T, NP, 1), jnp.float32),
            jax.ShapeDtypeStruct((T, NP, F), jnp.float32),
        ],
    )(cnt2, x_pad)


def _passA_body(s_ref, z_ref, dinv_ref, bht_ref, w_ref, b_ref,
                h_ref, s1_ref, s2_ref, cnt_ref):
    n = pl.program_id(1)
    dinv = dinv_ref[0]
    stot = s_ref[0, 0] + s_ref[1, 0] - z_ref[0]
    xin = jnp.where(dinv > 0, stot, 0.0) * dinv
    h = jnp.dot(xin, w_ref[...], preferred_element_type=jnp.float32) + b_ref[...]
    h_ref[0] = h
    bht = bht_ref[0]
    p1 = jnp.dot(bht, h, preferred_element_type=jnp.float32)
    p2 = jnp.dot(bht, h * h, preferred_element_type=jnp.float32)
    pc = jnp.sum(bht, axis=1, keepdims=True) * jnp.ones((1, 8), jnp.float32)

    @pl.when(n == 0)
    def _():
        s1_ref[0] = p1
        s2_ref[0] = p2
        cnt_ref[0] = pc

    @pl.when(n != 0)
    def _():
        s1_ref[0] += p1
        s2_ref[0] += p2
        cnt_ref[0] += pc


def _passA(s_all, z, dinv, bht, w, b2d, ho):
    return pl.pallas_call(
        _passA_body,
        grid=(T, NBLK),
        in_specs=[
            pl.BlockSpec((2, 1, NB, F), lambda t, n: (0, t, n, 0)),
            pl.BlockSpec((1, NB, F), lambda t, n: (t, n, 0)),
            pl.BlockSpec((1, NB, 1), lambda t, n: (t, n, 0)),
            pl.BlockSpec((1, G, NB), lambda t, n: (t, 0, n)),
            pl.BlockSpec((F, ho), lambda t, n: (0, 0)),
            pl.BlockSpec((1, ho), lambda t, n: (0, 0)),
        ],
        out_specs=[
            pl.BlockSpec((1, NB, ho), lambda t, n: (t, n, 0)),
            pl.BlockSpec((1, G, ho), lambda t, n: (t, 0, 0)),
            pl.BlockSpec((1, G, ho), lambda t, n: (t, 0, 0)),
            pl.BlockSpec((1, G, 8), lambda t, n: (t, 0, 0)),
        ],
        out_shape=[
            jax.ShapeDtypeStruct((T, NP, ho), jnp.float32),
            jax.ShapeDtypeStruct((T, G, ho), jnp.float32),
            jax.ShapeDtypeStruct((T, G, ho), jnp.float32),
            jax.ShapeDtypeStruct((T, G, 8), jnp.float32),
        ],
    )(s_all, z, dinv, bht, w, b2d)


def _passB_mid_body(h_ref, mean_ref, rstd_ref, dinv_ref, bht_ref,
                    gm_ref, bt_ref, al_ref, z_ref):
    h = h_ref[0]
    bht = bht_ref[0]
    mb = lax.dot_general(bht, mean_ref[0], (((0,), (0,)), ((), ())),
                         preferred_element_type=jnp.float32)
    rb = lax.dot_general(bht, rstd_ref[0], (((0,), (0,)), ((), ())),
                         preferred_element_type=jnp.float32)
    x = jnp.maximum(gm_ref[...] * (h - al_ref[...] * mb) * rb + bt_ref[...], 0.0)
    z_ref[0] = dinv_ref[0] * x


def _passB_mid(h, mean, rstd, dinv, bht, gm2, bt2, al2, ho):
    return pl.pallas_call(
        _passB_mid_body,
        grid=(T, NBLK),
        in_specs=[
            pl.BlockSpec((1, NB, ho), lambda t, n: (t, n, 0)),
            pl.BlockSpec((1, G, ho), lambda t, n: (t, 0, 0)),
            pl.BlockSpec((1, G, ho), lambda t, n: (t, 0, 0)),
            pl.BlockSpec((1, NB, 1), lambda t, n: (t, n, 0)),
            pl.BlockSpec((1, G, NB), lambda t, n: (t, 0, n)),
            pl.BlockSpec((1, ho), lambda t, n: (0, 0)),
            pl.BlockSpec((1, ho), lambda t, n: (0, 0)),
            pl.BlockSpec((1, ho), lambda t, n: (0, 0)),
        ],
        out_specs=[pl.BlockSpec((1, NB, F), lambda t, n: (t, n, 0))],
        out_shape=[jax.ShapeDtypeStruct((T, NP, F), jnp.float32)],
    )(h, mean, rstd, dinv, bht, gm2, bt2, al2)[0]


def _passB_last_body(h_ref, mean_ref, rstd_ref, bht_ref,
                     gm_ref, bt_ref, al_ref, psum_ref, pmax_ref):
    n = pl.program_id(1)
    h = h_ref[0]
    bht = bht_ref[0]
    mb = lax.dot_general(bht, mean_ref[0], (((0,), (0,)), ((), ())),
                         preferred_element_type=jnp.float32)
    rb = lax.dot_general(bht, rstd_ref[0], (((0,), (0,)), ((), ())),
                         preferred_element_type=jnp.float32)
    x = jnp.maximum(gm_ref[...] * (h - al_ref[...] * mb) * rb + bt_ref[...], 0.0)
    ps = jnp.dot(bht, x, preferred_element_type=jnp.float32)
    neg = jnp.float32(-jnp.inf)
    pm = jnp.stack([jnp.max(jnp.where(bht[g][:, None] > 0, x, neg), axis=0)
                    for g in range(G)], axis=0)

    @pl.when(n == 0)
    def _():
        psum_ref[0] = ps
        pmax_ref[0] = pm

    @pl.when(n != 0)
    def _():
        psum_ref[0] += ps
        pmax_ref[0] = jnp.maximum(pmax_ref[0], pm)


def _passB_last(h, mean, rstd, bht, gm2, bt2, al2, ho):
    return pl.pallas_call(
        _passB_last_body,
        grid=(T, NBLK),
        in_specs=[
            pl.BlockSpec((1, NB, ho), lambda t, n: (t, n, 0)),
            pl.BlockSpec((1, G, ho), lambda t, n: (t, 0, 0)),
            pl.BlockSpec((1, G, ho), lambda t, n: (t, 0, 0)),
            pl.BlockSpec((1, G, NB), lambda t, n: (t, 0, n)),
            pl.BlockSpec((1, ho), lambda t, n: (0, 0)),
            pl.BlockSpec((1, ho), lambda t, n: (0, 0)),
            pl.BlockSpec((1, ho), lambda t, n: (0, 0)),
        ],
        out_specs=[
            pl.BlockSpec((1, G, ho), lambda t, n: (t, 0, 0)),
            pl.BlockSpec((1, G, ho), lambda t, n: (t, 0, 0)),
        ],
        out_shape=[
            jax.ShapeDtypeStruct((T, G, ho), jnp.float32),
            jax.ShapeDtypeStruct((T, G, ho), jnp.float32),
        ],
    )(h, mean, rstd, bht, gm2, bt2, al2)


def _moments_to_norm(s1, s2, cnt8, alpha):
    cnt = cnt8[:, :, :1]
    mean = s1 / cnt
    eh2 = s2 / cnt
    var = eh2 - (alpha * (2.0 - alpha)) * mean * mean
    rstd = lax.rsqrt(var + 1e-5)
    return mean, rstd, cnt


def kernel(x_seq, edge_index_seq, batch_seq, W0, b0, gamma0, beta0, alpha0,
           W1, b1, gamma1, beta1, alpha1, W2, b2, gamma2, beta2, alpha2):
    f32 = jnp.float32
    pad_n = EPAD - E
    pad_rows = (jnp.arange(pad_n, dtype=jnp.int32) % ND)
    src = jnp.concatenate(
        [edge_index_seq[:, 0].astype(jnp.int32),
         jnp.broadcast_to(pad_rows, (T, pad_n))], axis=1)
    dst = jnp.concatenate(
        [edge_index_seq[:, 1].astype(jnp.int32),
         jnp.broadcast_to(N + pad_rows, (T, pad_n))], axis=1)
    dst_w = dst.reshape(T, NCORES, NTILES, KW, WIN)
    t_off = (jnp.arange(T, dtype=jnp.int32) * NP)[:, None, None, None, None]
    src_g = src.reshape(T, NCORES, NTILES, KW, WIN) + t_off
    toff = (jnp.arange(T, dtype=jnp.int32) * ACC)[:, None]
    deg_idx = (dst + toff).reshape(NCORES, NTILES, KD, WIN)
    zeros_flat = jnp.zeros((DEGL,), f32)

    bht = (batch_seq[:, None, :] == jnp.arange(G, dtype=batch_seq.dtype)[None, :, None]
           ).astype(f32)
    bht = jnp.pad(bht, ((0, 0), (0, 0), (0, NP - N)))
    x_pad = jnp.pad(x_seq, ((0, 0), (0, NP - N), (0, 0)))

    parts = _deg_counts(deg_idx, zeros_flat)
    cnt2 = parts.reshape(NCORES, T, ACC)[:, :, :N]
    cnt2 = jnp.pad(cnt2, ((0, 0), (0, 0), (0, NP - N)))[..., None]

    dinv, z = _prep(cnt2, x_pad)

    params = [(W0, b0, gamma0, beta0, alpha0),
              (W1, b1, gamma1, beta1, alpha1),
              (W2, b2, gamma2, beta2, alpha2)]
    for li, (W, b, gm, bt, al) in enumerate(params):
        ho = W.shape[1]
        s_all = _propagate(z, src_g, dst_w)
        h, s1, s2, cnt8 = _passA(s_all, z, dinv, bht, W, b.reshape(1, ho), ho)
        mean, rstd, cnt = _moments_to_norm(s1, s2, cnt8, al)
        if li < 2:
            z = _passB_mid(h, mean, rstd, dinv, bht, gm.reshape(1, ho),
                           bt.reshape(1, ho), al.reshape(1, ho), ho)
        else:
            psum, pmax = _passB_last(h, mean, rstd, bht, gm.reshape(1, ho),
                                     bt.reshape(1, ho), al.reshape(1, ho), ho)
    out = jnp.concatenate([psum / cnt, pmax], axis=-1)
    return jnp.mean(out, axis=0)


# speedup vs baseline: 21.7961x; 1.0696x over previous
import functools
import jax
import jax.numpy as jnp
from jax import lax
from jax.experimental import pallas as pl
from jax.experimental.pallas import tpu as pltpu
from jax.experimental.pallas import tpu_sc as plsc

T, N, E, F, G = 4, 10000, 320000, 128, 16
NTILES = 16
NCORES = 2
WIN = 128
UNROLL = 8
IDXC = 8
EPAD = 32 * WIN * 80
KW = EPAD // (NCORES * NTILES * WIN)
ND = 80
NP = 10240
ROWS_T = NP // NTILES
NB = 1024
NBLK = NP // NB
KD = T * EPAD // (NCORES * NTILES * WIN)
ACC = N + ND
DEGL = T * ACC


def _sc_mesh():
    return plsc.VectorSubcoreMesh(core_axis_name="c", subcore_axis_name="s")


def _deg_body(idx_hbm, zeros_hbm, out_hbm, idx_v, ones_v, bounce, acc_sp):
    c = lax.axis_index("c")
    s = lax.axis_index("s")
    for i in range(8):
        ones_v[pl.ds(i * 16, 16)] = jnp.ones((16,), jnp.float32)
    seg = DEGL // NTILES
    pltpu.sync_copy(zeros_hbm.at[pl.ds(s * seg, seg)], bounce)
    pltpu.sync_copy(bounce, acc_sp.at[pl.ds(s * seg, seg)])
    pltpu.sync_copy(idx_hbm.at[c, s], idx_v)
    plsc.subcore_barrier()

    def chunk(k, carry):
        for j in range(UNROLL):
            pltpu.sync_copy(ones_v, acc_sp.at[idx_v.at[k * UNROLL + j]], add=True)
        return carry

    lax.fori_loop(0, KD // UNROLL, chunk, 0)
    plsc.subcore_barrier()
    pltpu.sync_copy(acc_sp.at[pl.ds(s * seg, seg)], bounce)
    pltpu.sync_copy(bounce, out_hbm.at[pl.ds(c * DEGL + s * seg, seg)])


def _deg_counts(idx_w, zeros_flat):
    seg = DEGL // NTILES
    k = pl.kernel(
        _deg_body,
        out_type=jax.ShapeDtypeStruct((NCORES * DEGL,), jnp.float32),
        mesh=_sc_mesh(),
        scratch_types=[
            pltpu.VMEM((KD, WIN), jnp.int32),
            pltpu.VMEM((WIN,), jnp.float32),
            pltpu.VMEM((seg,), jnp.float32),
            pltpu.VMEM_SHARED((DEGL,), jnp.float32),
        ],
    )
    return k(idx_w, zeros_flat)


NCHUNK = KW // IDXC


def _scat_body(z_hbm, idx_hbm, out_hbm, ibuf, buf0, buf1,
               isem, gsem0, gsem1, ssem0, ssem1, acc_sp):
    c = lax.axis_index("c")
    s = lax.axis_index("s")
    gbufs = (buf0, buf1)
    gsems = (gsem0, gsem1)
    ssems = (ssem0, ssem1)
    nstage = ROWS_T // WIN
    for t in range(T):
        def stage_in(r, carry):
            rows = pl.ds(t * NP + s * ROWS_T + r * WIN, WIN)
            pltpu.sync_copy(z_hbm.at[rows], buf0)
            pltpu.sync_copy(buf0, acc_sp.at[pl.ds(s * ROWS_T + r * WIN, WIN)])
            return carry

        lax.fori_loop(0, nstage, stage_in, 0)
        plsc.subcore_barrier()

        pltpu.async_copy(idx_hbm.at[t, c, s, 0], ibuf.at[0], isem)

        def idx_chunk(kc, carry):
            par = lax.rem(kc, 2)
            pltpu.make_async_copy(idx_hbm.at[t, c, s, 0], ibuf.at[0], isem).wait()
            nkc = jnp.minimum(kc + 1, NCHUNK - 1)
            pltpu.async_copy(idx_hbm.at[t, c, s, nkc],
                             ibuf.at[lax.rem(kc + 1, 2)], isem)
            sd = [None, None]
            g = pltpu.async_copy(z_hbm.at[ibuf.at[par, 0, 0]], buf0, gsem0)
            for j in range(IDXC):
                if j + 1 < IDXC:
                    if sd[(j + 1) % 2] is not None:
                        sd[(j + 1) % 2].wait()
                    gn = pltpu.async_copy(z_hbm.at[ibuf.at[par, 0, j + 1]],
                                          gbufs[(j + 1) % 2], gsems[(j + 1) % 2])
                g.wait()
                sd[j % 2] = pltpu.async_copy(gbufs[j % 2],
                                             acc_sp.at[ibuf.at[par, 1, j]],
                                             ssems[j % 2], add=True)
                if j + 1 < IDXC:
                    g = gn
            sd[0].wait()
            sd[1].wait()
            return carry

        lax.fori_loop(0, NCHUNK, idx_chunk, 0)
        pltpu.make_async_copy(idx_hbm.at[t, c, s, 0], ibuf.at[0], isem).wait()
        plsc.subcore_barrier()

        def stage_out(r, carry):
            rows = pl.ds(s * ROWS_T + r * WIN, WIN)
            pltpu.sync_copy(acc_sp.at[rows], buf0)
            pltpu.sync_copy(buf0, out_hbm.at[c, t, rows])
            return carry

        lax.fori_loop(0, nstage, stage_out, 0)
        plsc.subcore_barrier()


def _propagate(z_all, idx_w):
    k = pl.kernel(
        _scat_body,
        out_type=jax.ShapeDtypeStruct((NCORES, T, NP, F), jnp.float32),
        mesh=_sc_mesh(),
        scratch_types=[
            pltpu.VMEM((2, 2, IDXC, WIN), jnp.int32),
            pltpu.VMEM((WIN, F), jnp.float32),
            pltpu.VMEM((WIN, F), jnp.float32),
            pltpu.SemaphoreType.DMA,
            pltpu.SemaphoreType.DMA,
            pltpu.SemaphoreType.DMA,
            pltpu.SemaphoreType.DMA,
            pltpu.SemaphoreType.DMA,
            pltpu.VMEM_SHARED((NP, F), jnp.float32),
        ],
    )
    return k(z_all.reshape(T * NP, F), idx_w)


def _prep_body(cnt_ref, x_ref, dinv_ref, z_ref):
    n = pl.program_id(1)
    rows = n * NB + lax.broadcasted_iota(jnp.int32, (NB, 1), 0)
    c = cnt_ref[0, 0] + cnt_ref[1, 0]
    dinv = jnp.where(rows < N, lax.rsqrt(1.0 + c), 0.0)
    dinv_ref[0] = dinv
    z_ref[0] = dinv * x_ref[0]


def _prep(cnt2, x_pad):
    return pl.pallas_call(
        _prep_body,
        grid=(T, NBLK),
        in_specs=[
            pl.BlockSpec((2, 1, NB, 1), lambda t, n: (0, t, n, 0)),
            pl.BlockSpec((1, NB, F), lambda t, n: (t, n, 0)),
        ],
        out_specs=[
            pl.BlockSpec((1, NB, 1), lambda t, n: (t, n, 0)),
            pl.BlockSpec((1, NB, F), lambda t, n: (t, n, 0)),
        ],
        out_shape=[
            jax.ShapeDtypeStruct((T, NP, 1), jnp.float32),
            jax.ShapeDtypeStruct((T, NP, F), jnp.float32),
        ],
    )(cnt2, x_pad)


def _passA_body(s_ref, z_ref, dinv_ref, bht_ref, w_ref, b_ref,
                h_ref, s1_ref, s2_ref, cnt_ref):
    n = pl.program_id(1)
    dinv = dinv_ref[0]
    stot = s_ref[0, 0] + s_ref[1, 0] - z_ref[0]
    xin = jnp.where(dinv > 0, stot, 0.0) * dinv
    h = jnp.dot(xin, w_ref[...], preferred_element_type=jnp.float32) + b_ref[...]
    h_ref[0] = h
    bht = bht_ref[0]
    p1 = jnp.dot(bht, h, preferred_element_type=jnp.float32)
    p2 = jnp.dot(bht, h * h, preferred_element_type=jnp.float32)
    pc = jnp.sum(bht, axis=1, keepdims=True) * jnp.ones((1, 8), jnp.float32)

    @pl.when(n == 0)
    def _():
        s1_ref[0] = p1
        s2_ref[0] = p2
        cnt_ref[0] = pc

    @pl.when(n != 0)
    def _():
        s1_ref[0] += p1
        s2_ref[0] += p2
        cnt_ref[0] += pc


def _passA(s_all, z, dinv, bht, w, b2d, ho):
    return pl.pallas_call(
        _passA_body,
        grid=(T, NBLK),
        in_specs=[
            pl.BlockSpec((2, 1, NB, F), lambda t, n: (0, t, n, 0)),
            pl.BlockSpec((1, NB, F), lambda t, n: (t, n, 0)),
            pl.BlockSpec((1, NB, 1), lambda t, n: (t, n, 0)),
            pl.BlockSpec((1, G, NB), lambda t, n: (t, 0, n)),
            pl.BlockSpec((F, ho), lambda t, n: (0, 0)),
            pl.BlockSpec((1, ho), lambda t, n: (0, 0)),
        ],
        out_specs=[
            pl.BlockSpec((1, NB, ho), lambda t, n: (t, n, 0)),
            pl.BlockSpec((1, G, ho), lambda t, n: (t, 0, 0)),
            pl.BlockSpec((1, G, ho), lambda t, n: (t, 0, 0)),
            pl.BlockSpec((1, G, 8), lambda t, n: (t, 0, 0)),
        ],
        out_shape=[
            jax.ShapeDtypeStruct((T, NP, ho), jnp.float32),
            jax.ShapeDtypeStruct((T, G, ho), jnp.float32),
            jax.ShapeDtypeStruct((T, G, ho), jnp.float32),
            jax.ShapeDtypeStruct((T, G, 8), jnp.float32),
        ],
    )(s_all, z, dinv, bht, w, b2d)


def _passB_mid_body(h_ref, mean_ref, rstd_ref, dinv_ref, bht_ref,
                    gm_ref, bt_ref, al_ref, z_ref):
    h = h_ref[0]
    bht = bht_ref[0]
    mb = lax.dot_general(bht, mean_ref[0], (((0,), (0,)), ((), ())),
                         preferred_element_type=jnp.float32)
    rb = lax.dot_general(bht, rstd_ref[0], (((0,), (0,)), ((), ())),
                         preferred_element_type=jnp.float32)
    x = jnp.maximum(gm_ref[...] * (h - al_ref[...] * mb) * rb + bt_ref[...], 0.0)
    z_ref[0] = dinv_ref[0] * x


def _passB_mid(h, mean, rstd, dinv, bht, gm2, bt2, al2, ho):
    return pl.pallas_call(
        _passB_mid_body,
        grid=(T, NBLK),
        in_specs=[
            pl.BlockSpec((1, NB, ho), lambda t, n: (t, n, 0)),
            pl.BlockSpec((1, G, ho), lambda t, n: (t, 0, 0)),
            pl.BlockSpec((1, G, ho), lambda t, n: (t, 0, 0)),
            pl.BlockSpec((1, NB, 1), lambda t, n: (t, n, 0)),
            pl.BlockSpec((1, G, NB), lambda t, n: (t, 0, n)),
            pl.BlockSpec((1, ho), lambda t, n: (0, 0)),
            pl.BlockSpec((1, ho), lambda t, n: (0, 0)),
            pl.BlockSpec((1, ho), lambda t, n: (0, 0)),
        ],
        out_specs=[pl.BlockSpec((1, NB, F), lambda t, n: (t, n, 0))],
        out_shape=[jax.ShapeDtypeStruct((T, NP, F), jnp.float32)],
    )(h, mean, rstd, dinv, bht, gm2, bt2, al2)[0]


def _passB_last_body(h_ref, mean_ref, rstd_ref, bht_ref,
                     gm_ref, bt_ref, al_ref, psum_ref, pmax_ref):
    n = pl.program_id(1)
    h = h_ref[0]
    bht = bht_ref[0]
    mb = lax.dot_general(bht, mean_ref[0], (((0,), (0,)), ((), ())),
                         preferred_element_type=jnp.float32)
    rb = lax.dot_general(bht, rstd_ref[0], (((0,), (0,)), ((), ())),
                         preferred_element_type=jnp.float32)
    x = jnp.maximum(gm_ref[...] * (h - al_ref[...] * mb) * rb + bt_ref[...], 0.0)
    ps = jnp.dot(bht, x, preferred_element_type=jnp.float32)
    neg = jnp.float32(-jnp.inf)
    pm = jnp.stack([jnp.max(jnp.where(bht[g][:, None] > 0, x, neg), axis=0)
                    for g in range(G)], axis=0)

    @pl.when(n == 0)
    def _():
        psum_ref[0] = ps
        pmax_ref[0] = pm

    @pl.when(n != 0)
    def _():
        psum_ref[0] += ps
        pmax_ref[0] = jnp.maximum(pmax_ref[0], pm)


def _passB_last(h, mean, rstd, bht, gm2, bt2, al2, ho):
    return pl.pallas_call(
        _passB_last_body,
        grid=(T, NBLK),
        in_specs=[
            pl.BlockSpec((1, NB, ho), lambda t, n: (t, n, 0)),
            pl.BlockSpec((1, G, ho), lambda t, n: (t, 0, 0)),
            pl.BlockSpec((1, G, ho), lambda t, n: (t, 0, 0)),
            pl.BlockSpec((1, G, NB), lambda t, n: (t, 0, n)),
            pl.BlockSpec((1, ho), lambda t, n: (0, 0)),
            pl.BlockSpec((1, ho), lambda t, n: (0, 0)),
            pl.BlockSpec((1, ho), lambda t, n: (0, 0)),
        ],
        out_specs=[
            pl.BlockSpec((1, G, ho), lambda t, n: (t, 0, 0)),
            pl.BlockSpec((1, G, ho), lambda t, n: (t, 0, 0)),
        ],
        out_shape=[
            jax.ShapeDtypeStruct((T, G, ho), jnp.float32),
            jax.ShapeDtypeStruct((T, G, ho), jnp.float32),
        ],
    )(h, mean, rstd, bht, gm2, bt2, al2)


def _moments_to_norm(s1, s2, cnt8, alpha):
    cnt = cnt8[:, :, :1]
    mean = s1 / cnt
    eh2 = s2 / cnt
    var = eh2 - (alpha * (2.0 - alpha)) * mean * mean
    rstd = lax.rsqrt(var + 1e-5)
    return mean, rstd, cnt


def kernel(x_seq, edge_index_seq, batch_seq, W0, b0, gamma0, beta0, alpha0,
           W1, b1, gamma1, beta1, alpha1, W2, b2, gamma2, beta2, alpha2):
    f32 = jnp.float32
    pad_n = EPAD - E
    pad_rows = (jnp.arange(pad_n, dtype=jnp.int32) % ND)
    src = jnp.concatenate(
        [edge_index_seq[:, 0].astype(jnp.int32),
         jnp.broadcast_to(pad_rows, (T, pad_n))], axis=1)
    dst = jnp.concatenate(
        [edge_index_seq[:, 1].astype(jnp.int32),
         jnp.broadcast_to(N + pad_rows, (T, pad_n))], axis=1)
    dst_w = dst.reshape(T, NCORES, NTILES, NCHUNK, IDXC, WIN)
    t_off = jnp.arange(T, dtype=jnp.int32).reshape(T, 1, 1, 1, 1, 1) * NP
    src_g = src.reshape(T, NCORES, NTILES, NCHUNK, IDXC, WIN) + t_off
    idx_w = jnp.stack([src_g, dst_w], axis=4)
    toff = (jnp.arange(T, dtype=jnp.int32) * ACC)[:, None]
    deg_idx = (dst + toff).reshape(NCORES, NTILES, KD, WIN)
    zeros_flat = jnp.zeros((DEGL,), f32)

    bht = (batch_seq[:, None, :] == jnp.arange(G, dtype=batch_seq.dtype)[None, :, None]
           ).astype(f32)
    bht = jnp.pad(bht, ((0, 0), (0, 0), (0, NP - N)))
    x_pad = jnp.pad(x_seq, ((0, 0), (0, NP - N), (0, 0)))

    parts = _deg_counts(deg_idx, zeros_flat)
    cnt2 = parts.reshape(NCORES, T, ACC)[:, :, :N]
    cnt2 = jnp.pad(cnt2, ((0, 0), (0, 0), (0, NP - N)))[..., None]

    dinv, z = _prep(cnt2, x_pad)

    params = [(W0, b0, gamma0, beta0, alpha0),
              (W1, b1, gamma1, beta1, alpha1),
              (W2, b2, gamma2, beta2, alpha2)]
    for li, (W, b, gm, bt, al) in enumerate(params):
        ho = W.shape[1]
        s_all = _propagate(z, idx_w)
        h, s1, s2, cnt8 = _passA(s_all, z, dinv, bht, W, b.reshape(1, ho), ho)
        mean, rstd, cnt = _moments_to_norm(s1, s2, cnt8, al)
        if li < 2:
            z = _passB_mid(h, mean, rstd, dinv, bht, gm.reshape(1, ho),
                           bt.reshape(1, ho), al.reshape(1, ho), ho)
        else:
            psum, pmax = _passB_last(h, mean, rstd, bht, gm.reshape(1, ho),
                                     bt.reshape(1, ho), al.reshape(1, ho), ho)
    out = jnp.concatenate([psum / cnt, pmax], axis=-1)
    return jnp.mean(out, axis=0)


# speedup vs baseline: 22.1023x; 1.0140x over previous
import functools
import jax
import jax.numpy as jnp
from jax import lax
from jax.experimental import pallas as pl
from jax.experimental.pallas import tpu as pltpu
from jax.experimental.pallas import tpu_sc as plsc

T, N, E, F, G = 4, 10000, 320000, 128, 16
NTILES = 16
NCORES = 2
WIN = 128
UNROLL = 8
IDXC = 8
EPAD = 32 * WIN * 80
KW = EPAD // (NCORES * NTILES * WIN)
ND = 80
NP = 10240
ROWS_T = NP // NTILES
NB = 1024
NBLK = NP // NB
KD = T * EPAD // (NCORES * NTILES * WIN)
ACC = N + ND
DEGL = T * ACC


def _sc_mesh():
    return plsc.VectorSubcoreMesh(core_axis_name="c", subcore_axis_name="s")


def _deg_body(idx_hbm, zeros_hbm, out_hbm, idx_v, ones_v, bounce, acc_sp):
    c = lax.axis_index("c")
    s = lax.axis_index("s")
    for i in range(8):
        ones_v[pl.ds(i * 16, 16)] = jnp.ones((16,), jnp.float32)
    seg = DEGL // NTILES
    pltpu.sync_copy(zeros_hbm.at[pl.ds(s * seg, seg)], bounce)
    pltpu.sync_copy(bounce, acc_sp.at[pl.ds(s * seg, seg)])
    pltpu.sync_copy(idx_hbm.at[c, s], idx_v)
    plsc.subcore_barrier()

    def chunk(k, carry):
        for j in range(UNROLL):
            pltpu.sync_copy(ones_v, acc_sp.at[idx_v.at[k * UNROLL + j]], add=True)
        return carry

    lax.fori_loop(0, KD // UNROLL, chunk, 0)
    plsc.subcore_barrier()
    pltpu.sync_copy(acc_sp.at[pl.ds(s * seg, seg)], bounce)
    pltpu.sync_copy(bounce, out_hbm.at[pl.ds(c * DEGL + s * seg, seg)])


def _deg_counts(idx_w, zeros_flat):
    seg = DEGL // NTILES
    k = pl.kernel(
        _deg_body,
        out_type=jax.ShapeDtypeStruct((NCORES * DEGL,), jnp.float32),
        mesh=_sc_mesh(),
        scratch_types=[
            pltpu.VMEM((KD, WIN), jnp.int32),
            pltpu.VMEM((WIN,), jnp.float32),
            pltpu.VMEM((seg,), jnp.float32),
            pltpu.VMEM_SHARED((DEGL,), jnp.float32),
        ],
    )
    return k(idx_w, zeros_flat)


NCHUNK = KW // IDXC


def _scat_body(z_hbm, idx_hbm, out_hbm, ibuf, buf0, buf1,
               isem, gsem0, gsem1, ssem0, ssem1, acc_sp):
    c = lax.axis_index("c")
    s = lax.axis_index("s")
    gbufs = (buf0, buf1)
    gsems = (gsem0, gsem1)
    ssems = (ssem0, ssem1)
    nstage = ROWS_T // WIN
    for t in range(T):
        def stage_in(r, carry):
            rows = pl.ds(t * NP + s * ROWS_T + r * WIN, WIN)
            pltpu.sync_copy(z_hbm.at[rows], buf0)
            pltpu.sync_copy(buf0, acc_sp.at[pl.ds(s * ROWS_T + r * WIN, WIN)])
            return carry

        lax.fori_loop(0, nstage, stage_in, 0)
        plsc.subcore_barrier()

        pltpu.async_copy(idx_hbm.at[t, c, s, 0], ibuf.at[0], isem)

        def idx_chunk(kc, carry):
            par = lax.rem(kc, 2)
            pltpu.make_async_copy(idx_hbm.at[t, c, s, 0], ibuf.at[0], isem).wait()
            nkc = jnp.minimum(kc + 1, NCHUNK - 1)
            pltpu.async_copy(idx_hbm.at[t, c, s, nkc],
                             ibuf.at[lax.rem(kc + 1, 2)], isem)
            @pl.when(kc > 0)
            def _():
                pltpu.make_async_copy(z_hbm.at[pl.ds(0, WIN)], buf0,
                                      ssem0).wait()

            sd = [None, None]
            g = pltpu.async_copy(z_hbm.at[ibuf.at[par, 0, 0]], buf0, gsem0)
            for j in range(IDXC):
                if j + 1 < IDXC:
                    p = (j + 1) % 2
                    if sd[p] is not None:
                        sd[p].wait()
                    else:
                        @pl.when(kc > 0)
                        def _():
                            pltpu.make_async_copy(z_hbm.at[pl.ds(0, WIN)],
                                                  gbufs[p], ssems[p]).wait()
                    gn = pltpu.async_copy(z_hbm.at[ibuf.at[par, 0, j + 1]],
                                          gbufs[p], gsems[p])
                g.wait()
                sd[j % 2] = pltpu.async_copy(gbufs[j % 2],
                                             acc_sp.at[ibuf.at[par, 1, j]],
                                             ssems[j % 2], add=True)
                if j + 1 < IDXC:
                    g = gn
            return carry

        lax.fori_loop(0, NCHUNK, idx_chunk, 0)
        pltpu.make_async_copy(idx_hbm.at[t, c, s, 0], ibuf.at[0], isem).wait()
        pltpu.make_async_copy(z_hbm.at[pl.ds(0, WIN)], buf0, ssem0).wait()
        pltpu.make_async_copy(z_hbm.at[pl.ds(0, WIN)], buf1, ssem1).wait()
        plsc.subcore_barrier()

        def stage_out(r, carry):
            rows = pl.ds(s * ROWS_T + r * WIN, WIN)
            pltpu.sync_copy(acc_sp.at[rows], buf0)
            pltpu.sync_copy(buf0, out_hbm.at[c, t, rows])
            return carry

        lax.fori_loop(0, nstage, stage_out, 0)
        plsc.subcore_barrier()


def _propagate(z_all, idx_w):
    k = pl.kernel(
        _scat_body,
        out_type=jax.ShapeDtypeStruct((NCORES, T, NP, F), jnp.float32),
        mesh=_sc_mesh(),
        scratch_types=[
            pltpu.VMEM((2, 2, IDXC, WIN), jnp.int32),
            pltpu.VMEM((WIN, F), jnp.float32),
            pltpu.VMEM((WIN, F), jnp.float32),
            pltpu.SemaphoreType.DMA,
            pltpu.SemaphoreType.DMA,
            pltpu.SemaphoreType.DMA,
            pltpu.SemaphoreType.DMA,
            pltpu.SemaphoreType.DMA,
            pltpu.VMEM_SHARED((NP, F), jnp.float32),
        ],
    )
    return k(z_all.reshape(T * NP, F), idx_w)


def _prep_body(cnt_ref, x_ref, dinv_ref, z_ref):
    n = pl.program_id(1)
    rows = n * NB + lax.broadcasted_iota(jnp.int32, (NB, 1), 0)
    c = cnt_ref[0, 0] + cnt_ref[1, 0]
    dinv = jnp.where(rows < N, lax.rsqrt(1.0 + c), 0.0)
    dinv_ref[0] = dinv
    z_ref[0] = dinv * x_ref[0]


def _prep(cnt2, x_pad):
    return pl.pallas_call(
        _prep_body,
        grid=(T, NBLK),
        in_specs=[
            pl.BlockSpec((2, 1, NB, 1), lambda t, n: (0, t, n, 0)),
            pl.BlockSpec((1, NB, F), lambda t, n: (t, n, 0)),
        ],
        out_specs=[
            pl.BlockSpec((1, NB, 1), lambda t, n: (t, n, 0)),
            pl.BlockSpec((1, NB, F), lambda t, n: (t, n, 0)),
        ],
        out_shape=[
            jax.ShapeDtypeStruct((T, NP, 1), jnp.float32),
            jax.ShapeDtypeStruct((T, NP, F), jnp.float32),
        ],
    )(cnt2, x_pad)


def _passA_body(s_ref, z_ref, dinv_ref, bht_ref, w_ref, b_ref,
                h_ref, s1_ref, s2_ref, cnt_ref):
    n = pl.program_id(1)
    dinv = dinv_ref[0]
    stot = s_ref[0, 0] + s_ref[1, 0] - z_ref[0]
    xin = jnp.where(dinv > 0, stot, 0.0) * dinv
    h = jnp.dot(xin, w_ref[...], preferred_element_type=jnp.float32) + b_ref[...]
    h_ref[0] = h
    bht = bht_ref[0]
    p1 = jnp.dot(bht, h, preferred_element_type=jnp.float32)
    p2 = jnp.dot(bht, h * h, preferred_element_type=jnp.float32)
    pc = jnp.sum(bht, axis=1, keepdims=True) * jnp.ones((1, 8), jnp.float32)

    @pl.when(n == 0)
    def _():
        s1_ref[0] = p1
        s2_ref[0] = p2
        cnt_ref[0] = pc

    @pl.when(n != 0)
    def _():
        s1_ref[0] += p1
        s2_ref[0] += p2
        cnt_ref[0] += pc


def _passA(s_all, z, dinv, bht, w, b2d, ho):
    return pl.pallas_call(
        _passA_body,
        grid=(T, NBLK),
        in_specs=[
            pl.BlockSpec((2, 1, NB, F), lambda t, n: (0, t, n, 0)),
            pl.BlockSpec((1, NB, F), lambda t, n: (t, n, 0)),
            pl.BlockSpec((1, NB, 1), lambda t, n: (t, n, 0)),
            pl.BlockSpec((1, G, NB), lambda t, n: (t, 0, n)),
            pl.BlockSpec((F, ho), lambda t, n: (0, 0)),
            pl.BlockSpec((1, ho), lambda t, n: (0, 0)),
        ],
        out_specs=[
            pl.BlockSpec((1, NB, ho), lambda t, n: (t, n, 0)),
            pl.BlockSpec((1, G, ho), lambda t, n: (t, 0, 0)),
            pl.BlockSpec((1, G, ho), lambda t, n: (t, 0, 0)),
            pl.BlockSpec((1, G, 8), lambda t, n: (t, 0, 0)),
        ],
        out_shape=[
            jax.ShapeDtypeStruct((T, NP, ho), jnp.float32),
            jax.ShapeDtypeStruct((T, G, ho), jnp.float32),
            jax.ShapeDtypeStruct((T, G, ho), jnp.float32),
            jax.ShapeDtypeStruct((T, G, 8), jnp.float32),
        ],
    )(s_all, z, dinv, bht, w, b2d)


def _passB_mid_body(h_ref, mean_ref, rstd_ref, dinv_ref, bht_ref,
                    gm_ref, bt_ref, al_ref, z_ref):
    h = h_ref[0]
    bht = bht_ref[0]
    mb = lax.dot_general(bht, mean_ref[0], (((0,), (0,)), ((), ())),
                         preferred_element_type=jnp.float32)
    rb = lax.dot_general(bht, rstd_ref[0], (((0,), (0,)), ((), ())),
                         preferred_element_type=jnp.float32)
    x = jnp.maximum(gm_ref[...] * (h - al_ref[...] * mb) * rb + bt_ref[...], 0.0)
    z_ref[0] = dinv_ref[0] * x


def _passB_mid(h, mean, rstd, dinv, bht, gm2, bt2, al2, ho):
    return pl.pallas_call(
        _passB_mid_body,
        grid=(T, NBLK),
        in_specs=[
            pl.BlockSpec((1, NB, ho), lambda t, n: (t, n, 0)),
            pl.BlockSpec((1, G, ho), lambda t, n: (t, 0, 0)),
            pl.BlockSpec((1, G, ho), lambda t, n: (t, 0, 0)),
            pl.BlockSpec((1, NB, 1), lambda t, n: (t, n, 0)),
            pl.BlockSpec((1, G, NB), lambda t, n: (t, 0, n)),
            pl.BlockSpec((1, ho), lambda t, n: (0, 0)),
            pl.BlockSpec((1, ho), lambda t, n: (0, 0)),
            pl.BlockSpec((1, ho), lambda t, n: (0, 0)),
        ],
        out_specs=[pl.BlockSpec((1, NB, F), lambda t, n: (t, n, 0))],
        out_shape=[jax.ShapeDtypeStruct((T, NP, F), jnp.float32)],
    )(h, mean, rstd, dinv, bht, gm2, bt2, al2)[0]


def _passB_last_body(h_ref, mean_ref, rstd_ref, bht_ref,
                     gm_ref, bt_ref, al_ref, psum_ref, pmax_ref):
    n = pl.program_id(1)
    h = h_ref[0]
    bht = bht_ref[0]
    mb = lax.dot_general(bht, mean_ref[0], (((0,), (0,)), ((), ())),
                         preferred_element_type=jnp.float32)
    rb = lax.dot_general(bht, rstd_ref[0], (((0,), (0,)), ((), ())),
                         preferred_element_type=jnp.float32)
    x = jnp.maximum(gm_ref[...] * (h - al_ref[...] * mb) * rb + bt_ref[...], 0.0)
    ps = jnp.dot(bht, x, preferred_element_type=jnp.float32)
    neg = jnp.float32(-jnp.inf)
    pm = jnp.stack([jnp.max(jnp.where(bht[g][:, None] > 0, x, neg), axis=0)
                    for g in range(G)], axis=0)

    @pl.when(n == 0)
    def _():
        psum_ref[0] = ps
        pmax_ref[0] = pm

    @pl.when(n != 0)
    def _():
        psum_ref[0] += ps
        pmax_ref[0] = jnp.maximum(pmax_ref[0], pm)


def _passB_last(h, mean, rstd, bht, gm2, bt2, al2, ho):
    return pl.pallas_call(
        _passB_last_body,
        grid=(T, NBLK),
        in_specs=[
            pl.BlockSpec((1, NB, ho), lambda t, n: (t, n, 0)),
            pl.BlockSpec((1, G, ho), lambda t, n: (t, 0, 0)),
            pl.BlockSpec((1, G, ho), lambda t, n: (t, 0, 0)),
            pl.BlockSpec((1, G, NB), lambda t, n: (t, 0, n)),
            pl.BlockSpec((1, ho), lambda t, n: (0, 0)),
            pl.BlockSpec((1, ho), lambda t, n: (0, 0)),
            pl.BlockSpec((1, ho), lambda t, n: (0, 0)),
        ],
        out_specs=[
            pl.BlockSpec((1, G, ho), lambda t, n: (t, 0, 0)),
            pl.BlockSpec((1, G, ho), lambda t, n: (t, 0, 0)),
        ],
        out_shape=[
            jax.ShapeDtypeStruct((T, G, ho), jnp.float32),
            jax.ShapeDtypeStruct((T, G, ho), jnp.float32),
        ],
    )(h, mean, rstd, bht, gm2, bt2, al2)


def _moments_to_norm(s1, s2, cnt8, alpha):
    cnt = cnt8[:, :, :1]
    mean = s1 / cnt
    eh2 = s2 / cnt
    var = eh2 - (alpha * (2.0 - alpha)) * mean * mean
    rstd = lax.rsqrt(var + 1e-5)
    return mean, rstd, cnt


def kernel(x_seq, edge_index_seq, batch_seq, W0, b0, gamma0, beta0, alpha0,
           W1, b1, gamma1, beta1, alpha1, W2, b2, gamma2, beta2, alpha2):
    f32 = jnp.float32
    pad_n = EPAD - E
    pad_rows = (jnp.arange(pad_n, dtype=jnp.int32) % ND)
    src = jnp.concatenate(
        [edge_index_seq[:, 0].astype(jnp.int32),
         jnp.broadcast_to(pad_rows, (T, pad_n))], axis=1)
    dst = jnp.concatenate(
        [edge_index_seq[:, 1].astype(jnp.int32),
         jnp.broadcast_to(N + pad_rows, (T, pad_n))], axis=1)
    dst_w = dst.reshape(T, NCORES, NTILES, NCHUNK, IDXC, WIN)
    t_off = jnp.arange(T, dtype=jnp.int32).reshape(T, 1, 1, 1, 1, 1) * NP
    src_g = src.reshape(T, NCORES, NTILES, NCHUNK, IDXC, WIN) + t_off
    idx_w = jnp.stack([src_g, dst_w], axis=4)
    toff = (jnp.arange(T, dtype=jnp.int32) * ACC)[:, None]
    deg_idx = (dst + toff).reshape(NCORES, NTILES, KD, WIN)
    zeros_flat = jnp.zeros((DEGL,), f32)

    bht = (batch_seq[:, None, :] == jnp.arange(G, dtype=batch_seq.dtype)[None, :, None]
           ).astype(f32)
    bht = jnp.pad(bht, ((0, 0), (0, 0), (0, NP - N)))
    x_pad = jnp.pad(x_seq, ((0, 0), (0, NP - N), (0, 0)))

    parts = _deg_counts(deg_idx, zeros_flat)
    cnt2 = parts.reshape(NCORES, T, ACC)[:, :, :N]
    cnt2 = jnp.pad(cnt2, ((0, 0), (0, 0), (0, NP - N)))[..., None]

    dinv, z = _prep(cnt2, x_pad)

    params = [(W0, b0, gamma0, beta0, alpha0),
              (W1, b1, gamma1, beta1, alpha1),
              (W2, b2, gamma2, beta2, alpha2)]
    for li, (W, b, gm, bt, al) in enumerate(params):
        ho = W.shape[1]
        s_all = _propagate(z, idx_w)
        h, s1, s2, cnt8 = _passA(s_all, z, dinv, bht, W, b.reshape(1, ho), ho)
        mean, rstd, cnt = _moments_to_norm(s1, s2, cnt8, al)
        if li < 2:
            z = _passB_mid(h, mean, rstd, dinv, bht, gm.reshape(1, ho),
                           bt.reshape(1, ho), al.reshape(1, ho), ho)
        else:
            psum, pmax = _passB_last(h, mean, rstd, bht, gm.reshape(1, ho),
                                     bt.reshape(1, ho), al.reshape(1, ho), ho)
    out = jnp.concatenate([psum / cnt, pmax], axis=-1)
    return jnp.mean(out, axis=0)


# speedup vs baseline: 22.5214x; 1.0190x over previous
import functools
import jax
import jax.numpy as jnp
from jax import lax
from jax.experimental import pallas as pl
from jax.experimental.pallas import tpu as pltpu
from jax.experimental.pallas import tpu_sc as plsc

T, N, E, F, G = 4, 10000, 320000, 128, 16
NTILES = 16
NCORES = 2
WIN = 128
UNROLL = 8
IDXC = 8
EPAD = 32 * WIN * 80
KW = EPAD // (NCORES * NTILES * WIN)
ND = 80
NP = 10240
ROWS_T = NP // NTILES
NB = 1024
NBLK = NP // NB
KD = T * EPAD // (NCORES * NTILES * WIN)
ACC = N + ND
DEGL = T * ACC


def _sc_mesh():
    return plsc.VectorSubcoreMesh(core_axis_name="c", subcore_axis_name="s")


def _deg_body(idx_hbm, zeros_hbm, out_hbm, idx_v, ones_v, bounce, acc_sp):
    c = lax.axis_index("c")
    s = lax.axis_index("s")
    for i in range(8):
        ones_v[pl.ds(i * 16, 16)] = jnp.ones((16,), jnp.float32)
    seg = DEGL // NTILES
    pltpu.sync_copy(zeros_hbm.at[pl.ds(s * seg, seg)], bounce)
    pltpu.sync_copy(bounce, acc_sp.at[pl.ds(s * seg, seg)])
    pltpu.sync_copy(idx_hbm.at[c, s], idx_v)
    plsc.subcore_barrier()

    def chunk(k, carry):
        for j in range(UNROLL):
            pltpu.sync_copy(ones_v, acc_sp.at[idx_v.at[k * UNROLL + j]], add=True)
        return carry

    lax.fori_loop(0, KD // UNROLL, chunk, 0)
    plsc.subcore_barrier()
    pltpu.sync_copy(acc_sp.at[pl.ds(s * seg, seg)], bounce)
    pltpu.sync_copy(bounce, out_hbm.at[pl.ds(c * DEGL + s * seg, seg)])


def _deg_counts(idx_w, zeros_flat):
    seg = DEGL // NTILES
    k = pl.kernel(
        _deg_body,
        out_type=jax.ShapeDtypeStruct((NCORES * DEGL,), jnp.float32),
        mesh=_sc_mesh(),
        scratch_types=[
            pltpu.VMEM((KD, WIN), jnp.int32),
            pltpu.VMEM((WIN,), jnp.float32),
            pltpu.VMEM((seg,), jnp.float32),
            pltpu.VMEM_SHARED((DEGL,), jnp.float32),
        ],
    )
    return k(idx_w, zeros_flat)


NCHUNK = KW // IDXC


def _scat_body(z_hbm, idx_hbm, out_hbm, ibuf, buf0, buf1,
               isem, gsem0, gsem1, ssem0, ssem1, acc_sp):
    c = lax.axis_index("c")
    s = lax.axis_index("s")
    gbufs = (buf0, buf1)
    gsems = (gsem0, gsem1)
    ssems = (ssem0, ssem1)
    nstage = ROWS_T // WIN
    for t in range(T):
        def stage_in(r, carry):
            rows = pl.ds(t * NP + s * ROWS_T + r * WIN, WIN)
            pltpu.sync_copy(z_hbm.at[rows], buf0)
            pltpu.sync_copy(buf0, acc_sp.at[pl.ds(s * ROWS_T + r * WIN, WIN)])
            return carry

        lax.fori_loop(0, nstage, stage_in, 0)
        plsc.subcore_barrier()

        pltpu.async_copy(idx_hbm.at[t, c, s, 0], ibuf.at[0], isem)

        def idx_chunk(kc, carry):
            par = lax.rem(kc, 2)
            pltpu.make_async_copy(idx_hbm.at[t, c, s, 0], ibuf.at[0], isem).wait()
            nkc = jnp.minimum(kc + 1, NCHUNK - 1)
            pltpu.async_copy(idx_hbm.at[t, c, s, nkc],
                             ibuf.at[lax.rem(kc + 1, 2)], isem)
            @pl.when(kc > 0)
            def _():
                pltpu.make_async_copy(z_hbm.at[pl.ds(0, WIN)], buf0,
                                      ssem0).wait()

            sd = [None, None]
            g = pltpu.async_copy(z_hbm.at[ibuf.at[par, 0, 0]], buf0, gsem0)
            for j in range(IDXC):
                if j + 1 < IDXC:
                    p = (j + 1) % 2
                    if sd[p] is not None:
                        sd[p].wait()
                    else:
                        @pl.when(kc > 0)
                        def _():
                            pltpu.make_async_copy(z_hbm.at[pl.ds(0, WIN)],
                                                  gbufs[p], ssems[p]).wait()
                    gn = pltpu.async_copy(z_hbm.at[ibuf.at[par, 0, j + 1]],
                                          gbufs[p], gsems[p])
                g.wait()
                sd[j % 2] = pltpu.async_copy(gbufs[j % 2],
                                             acc_sp.at[ibuf.at[par, 1, j]],
                                             ssems[j % 2], add=True)
                if j + 1 < IDXC:
                    g = gn
            return carry

        lax.fori_loop(0, NCHUNK, idx_chunk, 0)
        pltpu.make_async_copy(idx_hbm.at[t, c, s, 0], ibuf.at[0], isem).wait()
        pltpu.make_async_copy(z_hbm.at[pl.ds(0, WIN)], buf0, ssem0).wait()
        pltpu.make_async_copy(z_hbm.at[pl.ds(0, WIN)], buf1, ssem1).wait()
        plsc.subcore_barrier()

        def stage_out(r, carry):
            rows = pl.ds(s * ROWS_T + r * WIN, WIN)
            pltpu.sync_copy(acc_sp.at[rows], buf0)
            pltpu.sync_copy(buf0, out_hbm.at[c, t, rows])
            return carry

        lax.fori_loop(0, nstage, stage_out, 0)
        plsc.subcore_barrier()


def _propagate(z_all, idx_w):
    k = pl.kernel(
        _scat_body,
        out_type=jax.ShapeDtypeStruct((NCORES, T, NP, F), jnp.float32),
        mesh=_sc_mesh(),
        scratch_types=[
            pltpu.VMEM((2, 2, IDXC, WIN), jnp.int32),
            pltpu.VMEM((WIN, F), jnp.float32),
            pltpu.VMEM((WIN, F), jnp.float32),
            pltpu.SemaphoreType.DMA,
            pltpu.SemaphoreType.DMA,
            pltpu.SemaphoreType.DMA,
            pltpu.SemaphoreType.DMA,
            pltpu.SemaphoreType.DMA,
            pltpu.VMEM_SHARED((NP, F), jnp.float32),
        ],
    )
    return k(z_all.reshape(T * NP, F), idx_w)


def _prep_body(cnt_ref, x_ref, dinv_ref, z_ref):
    n = pl.program_id(1)
    rows = n * NB + lax.broadcasted_iota(jnp.int32, (NB, 1), 0)
    valid = rows < N
    c = cnt_ref[0, 0] + cnt_ref[1, 0]
    dinv = jnp.where(valid, lax.rsqrt(1.0 + c), 0.0)
    dinv_ref[0] = dinv
    z_ref[0] = dinv * jnp.where(valid, x_ref[0], 0.0)


def _prep(cnt2, x_seq):
    return pl.pallas_call(
        _prep_body,
        grid=(T, NBLK),
        in_specs=[
            pl.BlockSpec((2, 1, NB, 1), lambda t, n: (0, t, n, 0)),
            pl.BlockSpec((1, NB, F), lambda t, n: (t, n, 0)),
        ],
        out_specs=[
            pl.BlockSpec((1, NB, 1), lambda t, n: (t, n, 0)),
            pl.BlockSpec((1, NB, F), lambda t, n: (t, n, 0)),
        ],
        out_shape=[
            jax.ShapeDtypeStruct((T, NP, 1), jnp.float32),
            jax.ShapeDtypeStruct((T, NP, F), jnp.float32),
        ],
    )(cnt2, x_seq)


def _onehot(batch_ref, n):
    rows = n * NB + lax.broadcasted_iota(jnp.int32, (NB, G), 0)
    gids = lax.broadcasted_iota(jnp.int32, (NB, G), 1)
    return jnp.where((batch_ref[0] == gids) & (rows < N), 1.0, 0.0)


def _passA_body(s_ref, z_ref, dinv_ref, batch_ref, w_ref, b_ref,
                h_ref, s1_ref, s2_ref, cnt_ref):
    n = pl.program_id(1)
    dinv = dinv_ref[0]
    stot = s_ref[0, 0] + s_ref[1, 0] - z_ref[0]
    xin = jnp.where(dinv > 0, stot, 0.0) * dinv
    h = jnp.dot(xin, w_ref[...], preferred_element_type=jnp.float32) + b_ref[...]
    h_ref[0] = h
    bh = _onehot(batch_ref, n)
    cdim = (((0,), (0,)), ((), ()))
    p1 = lax.dot_general(bh, h, cdim, preferred_element_type=jnp.float32)
    p2 = lax.dot_general(bh, h * h, cdim, preferred_element_type=jnp.float32)
    pc = jnp.sum(bh, axis=0)[:, None] * jnp.ones((1, 8), jnp.float32)

    @pl.when(n == 0)
    def _():
        s1_ref[0] = p1
        s2_ref[0] = p2
        cnt_ref[0] = pc

    @pl.when(n != 0)
    def _():
        s1_ref[0] += p1
        s2_ref[0] += p2
        cnt_ref[0] += pc


def _passA(s_all, z, dinv, batch, w, b2d, ho):
    return pl.pallas_call(
        _passA_body,
        grid=(T, NBLK),
        in_specs=[
            pl.BlockSpec((2, 1, NB, F), lambda t, n: (0, t, n, 0)),
            pl.BlockSpec((1, NB, F), lambda t, n: (t, n, 0)),
            pl.BlockSpec((1, NB, 1), lambda t, n: (t, n, 0)),
            pl.BlockSpec((1, NB, 1), lambda t, n: (t, n, 0)),
            pl.BlockSpec((F, ho), lambda t, n: (0, 0)),
            pl.BlockSpec((1, ho), lambda t, n: (0, 0)),
        ],
        out_specs=[
            pl.BlockSpec((1, NB, ho), lambda t, n: (t, n, 0)),
            pl.BlockSpec((1, G, ho), lambda t, n: (t, 0, 0)),
            pl.BlockSpec((1, G, ho), lambda t, n: (t, 0, 0)),
            pl.BlockSpec((1, G, 8), lambda t, n: (t, 0, 0)),
        ],
        out_shape=[
            jax.ShapeDtypeStruct((T, NP, ho), jnp.float32),
            jax.ShapeDtypeStruct((T, G, ho), jnp.float32),
            jax.ShapeDtypeStruct((T, G, ho), jnp.float32),
            jax.ShapeDtypeStruct((T, G, 8), jnp.float32),
        ],
    )(s_all, z, dinv, batch, w, b2d)


def _passB_mid_body(h_ref, mean_ref, rstd_ref, dinv_ref, batch_ref,
                    gm_ref, bt_ref, al_ref, z_ref):
    h = h_ref[0]
    bh = _onehot(batch_ref, pl.program_id(1))
    mb = jnp.dot(bh, mean_ref[0], preferred_element_type=jnp.float32)
    rb = jnp.dot(bh, rstd_ref[0], preferred_element_type=jnp.float32)
    x = jnp.maximum(gm_ref[...] * (h - al_ref[...] * mb) * rb + bt_ref[...], 0.0)
    z_ref[0] = dinv_ref[0] * x


def _passB_mid(h, mean, rstd, dinv, batch, gm2, bt2, al2, ho):
    return pl.pallas_call(
        _passB_mid_body,
        grid=(T, NBLK),
        in_specs=[
            pl.BlockSpec((1, NB, ho), lambda t, n: (t, n, 0)),
            pl.BlockSpec((1, G, ho), lambda t, n: (t, 0, 0)),
            pl.BlockSpec((1, G, ho), lambda t, n: (t, 0, 0)),
            pl.BlockSpec((1, NB, 1), lambda t, n: (t, n, 0)),
            pl.BlockSpec((1, NB, 1), lambda t, n: (t, n, 0)),
            pl.BlockSpec((1, ho), lambda t, n: (0, 0)),
            pl.BlockSpec((1, ho), lambda t, n: (0, 0)),
            pl.BlockSpec((1, ho), lambda t, n: (0, 0)),
        ],
        out_specs=[pl.BlockSpec((1, NB, F), lambda t, n: (t, n, 0))],
        out_shape=[jax.ShapeDtypeStruct((T, NP, F), jnp.float32)],
    )(h, mean, rstd, dinv, batch, gm2, bt2, al2)[0]


def _passB_last_body(h_ref, mean_ref, rstd_ref, batch_ref,
                     gm_ref, bt_ref, al_ref, psum_ref, pmax_ref):
    n = pl.program_id(1)
    h = h_ref[0]
    bh = _onehot(batch_ref, n)
    mb = jnp.dot(bh, mean_ref[0], preferred_element_type=jnp.float32)
    rb = jnp.dot(bh, rstd_ref[0], preferred_element_type=jnp.float32)
    x = jnp.maximum(gm_ref[...] * (h - al_ref[...] * mb) * rb + bt_ref[...], 0.0)
    ps = lax.dot_general(bh, x, (((0,), (0,)), ((), ())),
                         preferred_element_type=jnp.float32)
    neg = jnp.float32(-jnp.inf)
    pm = jnp.stack([jnp.max(jnp.where(bh[:, g:g + 1] > 0, x, neg), axis=0)
                    for g in range(G)], axis=0)

    @pl.when(n == 0)
    def _():
        psum_ref[0] = ps
        pmax_ref[0] = pm

    @pl.when(n != 0)
    def _():
        psum_ref[0] += ps
        pmax_ref[0] = jnp.maximum(pmax_ref[0], pm)


def _passB_last(h, mean, rstd, batch, gm2, bt2, al2, ho):
    return pl.pallas_call(
        _passB_last_body,
        grid=(T, NBLK),
        in_specs=[
            pl.BlockSpec((1, NB, ho), lambda t, n: (t, n, 0)),
            pl.BlockSpec((1, G, ho), lambda t, n: (t, 0, 0)),
            pl.BlockSpec((1, G, ho), lambda t, n: (t, 0, 0)),
            pl.BlockSpec((1, NB, 1), lambda t, n: (t, n, 0)),
            pl.BlockSpec((1, ho), lambda t, n: (0, 0)),
            pl.BlockSpec((1, ho), lambda t, n: (0, 0)),
            pl.BlockSpec((1, ho), lambda t, n: (0, 0)),
        ],
        out_specs=[
            pl.BlockSpec((1, G, ho), lambda t, n: (t, 0, 0)),
            pl.BlockSpec((1, G, ho), lambda t, n: (t, 0, 0)),
        ],
        out_shape=[
            jax.ShapeDtypeStruct((T, G, ho), jnp.float32),
            jax.ShapeDtypeStruct((T, G, ho), jnp.float32),
        ],
    )(h, mean, rstd, batch, gm2, bt2, al2)


def _moments_to_norm(s1, s2, cnt8, alpha):
    cnt = cnt8[:, :, :1]
    mean = s1 / cnt
    eh2 = s2 / cnt
    var = eh2 - (alpha * (2.0 - alpha)) * mean * mean
    rstd = lax.rsqrt(var + 1e-5)
    return mean, rstd, cnt


def kernel(x_seq, edge_index_seq, batch_seq, W0, b0, gamma0, beta0, alpha0,
           W1, b1, gamma1, beta1, alpha1, W2, b2, gamma2, beta2, alpha2):
    f32 = jnp.float32
    pad_n = EPAD - E
    pad_rows = (jnp.arange(pad_n, dtype=jnp.int32) % ND)
    src = jnp.concatenate(
        [edge_index_seq[:, 0].astype(jnp.int32),
         jnp.broadcast_to(pad_rows, (T, pad_n))], axis=1)
    dst = jnp.concatenate(
        [edge_index_seq[:, 1].astype(jnp.int32),
         jnp.broadcast_to(N + pad_rows, (T, pad_n))], axis=1)
    dst_w = dst.reshape(T, NCORES, NTILES, NCHUNK, IDXC, WIN)
    t_off = jnp.arange(T, dtype=jnp.int32).reshape(T, 1, 1, 1, 1, 1) * NP
    src_g = src.reshape(T, NCORES, NTILES, NCHUNK, IDXC, WIN) + t_off
    idx_w = jnp.stack([src_g, dst_w], axis=4)
    toff = (jnp.arange(T, dtype=jnp.int32) * ACC)[:, None]
    deg_idx = (dst + toff).reshape(NCORES, NTILES, KD, WIN)
    zeros_flat = jnp.zeros((DEGL,), f32)

    batch32 = batch_seq.astype(jnp.int32).reshape(T, N, 1)

    parts = _deg_counts(deg_idx, zeros_flat)
    cnt2 = parts.reshape(NCORES, T, ACC)[:, :, :N]
    cnt2 = jnp.pad(cnt2, ((0, 0), (0, 0), (0, NP - N)))[..., None]

    dinv, z = _prep(cnt2, x_seq)

    params = [(W0, b0, gamma0, beta0, alpha0),
              (W1, b1, gamma1, beta1, alpha1),
              (W2, b2, gamma2, beta2, alpha2)]
    for li, (W, b, gm, bt, al) in enumerate(params):
        ho = W.shape[1]
        s_all = _propagate(z, idx_w)
        h, s1, s2, cnt8 = _passA(s_all, z, dinv, batch32, W,
                                 b.reshape(1, ho), ho)
        mean, rstd, cnt = _moments_to_norm(s1, s2, cnt8, al)
        if li < 2:
            z = _passB_mid(h, mean, rstd, dinv, batch32, gm.reshape(1, ho),
                           bt.reshape(1, ho), al.reshape(1, ho), ho)
        else:
            psum, pmax = _passB_last(h, mean, rstd, batch32, gm.reshape(1, ho),
                                     bt.reshape(1, ho), al.reshape(1, ho), ho)
    out = jnp.concatenate([psum / cnt, pmax], axis=-1)
    return jnp.mean(out, axis=0)
